# baseline (speedup 1.0000x reference)
import functools

import jax
import jax.numpy as jnp
from jax import lax
from jax.experimental import pallas as pl
from jax.experimental.pallas import tpu as pltpu
from jax.experimental.pallas import tpu_sc as plsc


def _pre_body(x_ref, w_ref, whh_t_ref, bhh_ref, m_ref, gh_ref):
    xb = x_ref[...]
    m_ref[...] = jnp.dot(xb, w_ref[...], preferred_element_type=jnp.float32)
    gh_ref[...] = (
        jnp.dot(xb, whh_t_ref[...], preferred_element_type=jnp.float32)
        + bhh_ref[...]
    )


def _pre_call(x, w, whh_t, bhh, bn):
    n, d = x.shape
    d3 = whh_t.shape[1]
    grid = n // bn
    return pl.pallas_call(
        _pre_body,
        grid=(grid,),
        in_specs=[
            pl.BlockSpec((bn, d), lambda i: (i, 0)),
            pl.BlockSpec((d, d), lambda i: (0, 0)),
            pl.BlockSpec((d, d3), lambda i: (0, 0)),
            pl.BlockSpec((1, d3), lambda i: (0, 0)),
        ],
        out_specs=[
            pl.BlockSpec((bn, d), lambda i: (i, 0)),
            pl.BlockSpec((bn, d3), lambda i: (i, 0)),
        ],
        out_shape=[
            jax.ShapeDtypeStruct((n, d), jnp.float32),
            jax.ShapeDtypeStruct((n, d3), jnp.float32),
        ],
    )(x, w, whh_t, bhh)


def _make_sc_seg_sum(n, d, e_pad, k, ngrp, gchunk):
    info = plsc.get_sparse_core_info()
    nc, ns = info.num_cores, info.num_subcores
    nw = nc * ns
    assert e_pad == nw * ngrp * gchunk * k
    rows_per_tile = (-(-n // ns) + 7) // 8 * 8
    npad = rows_per_tile * ns

    mesh = plsc.VectorSubcoreMesh(core_axis_name="c", subcore_axis_name="s")

    @functools.partial(
        pl.kernel,
        out_type=jax.ShapeDtypeStruct((nc, npad, d), jnp.float32),
        mesh=mesh,
        scratch_types=[
            pltpu.VMEM((gchunk, k), jnp.int32),
            pltpu.VMEM((gchunk, k), jnp.int32),
            pltpu.VMEM((k, d), jnp.float32),
            pltpu.VMEM((k, d), jnp.float32),
            pltpu.VMEM_SHARED((npad, d), jnp.float32),
            pltpu.SemaphoreType.DMA,
            pltpu.SemaphoreType.DMA,
        ],
    )
    def seg_sum(m_hbm, src_hbm, dst_hbm, zeros_hbm, out_hbm,
                src_g, dst_g, buf_a, buf_b, agg_sh, sem_a, sem_b):
        cid = lax.axis_index("c")
        sid = lax.axis_index("s")
        wid = sid * nc + cid

        row0 = sid * rows_per_tile
        pltpu.sync_copy(zeros_hbm, agg_sh.at[pl.ds(row0, rows_per_tile)])
        plsc.subcore_barrier()

        for g in range(ngrp):
            pltpu.sync_copy(src_hbm.at[wid, g], src_g)
            pltpu.sync_copy(dst_hbm.at[wid, g], dst_g)

            pltpu.async_copy(m_hbm.at[src_g.at[0]], buf_a, sem_a)
            pltpu.async_copy(m_hbm.at[src_g.at[1]], buf_b, sem_b)

            def pair(j, carry):
                i = j * 2
                pltpu.make_async_copy(
                    m_hbm.at[src_g.at[i]], buf_a, sem_a).wait()
                pltpu.sync_copy(buf_a, agg_sh.at[dst_g.at[i]], add=True)

                @pl.when(i + 2 < gchunk)
                def _():
                    pltpu.async_copy(m_hbm.at[src_g.at[i + 2]], buf_a, sem_a)

                pltpu.make_async_copy(
                    m_hbm.at[src_g.at[i + 1]], buf_b, sem_b).wait()
                pltpu.sync_copy(buf_b, agg_sh.at[dst_g.at[i + 1]], add=True)

                @pl.when(i + 3 < gchunk)
                def _():
                    pltpu.async_copy(m_hbm.at[src_g.at[i + 3]], buf_b, sem_b)

                return carry

            lax.fori_loop(0, gchunk // 2, pair, 0)

        plsc.subcore_barrier()
        pltpu.sync_copy(
            agg_sh.at[pl.ds(row0, rows_per_tile)],
            out_hbm.at[cid, pl.ds(row0, rows_per_tile)],
        )

    return seg_sum


def _post_body(p0_ref, p1_ref, x_ref, gh_ref, wih_t_ref, bih_ref, out_ref):
    d = x_ref.shape[1]
    agg = p0_ref[...] + p1_ref[...]
    gi = (
        jnp.dot(agg, wih_t_ref[...], preferred_element_type=jnp.float32)
        + bih_ref[...]
    )
    gh = gh_ref[...]
    xb = x_ref[...]
    r = jax.nn.sigmoid(gi[:, :d] + gh[:, :d])
    z = jax.nn.sigmoid(gi[:, d:2 * d] + gh[:, d:2 * d])
    nn = jnp.tanh(gi[:, 2 * d:] + r * gh[:, 2 * d:])
    h = (1.0 - z) * nn + z * xb
    out_ref[...] = xb + jnp.maximum(h, 0.0)


def _post_call(p0, p1, x, gh, wih_t, bih, bn):
    n, d = x.shape
    d3 = wih_t.shape[1]
    grid = n // bn
    return pl.pallas_call(
        _post_body,
        grid=(grid,),
        in_specs=[
            pl.BlockSpec((bn, d), lambda i: (i, 0)),
            pl.BlockSpec((bn, d), lambda i: (i, 0)),
            pl.BlockSpec((bn, d), lambda i: (i, 0)),
            pl.BlockSpec((bn, d3), lambda i: (i, 0)),
            pl.BlockSpec((d, d3), lambda i: (0, 0)),
            pl.BlockSpec((1, d3), lambda i: (0, 0)),
        ],
        out_specs=pl.BlockSpec((bn, d), lambda i: (i, 0)),
        out_shape=jax.ShapeDtypeStruct((n, d), jnp.float32),
    )(p0, p1, x, gh, wih_t, bih)


def kernel(x, edge_index, W, W_ih, W_hh, b_ih, b_hh):
    n, d = x.shape
    e = edge_index.shape[1]
    src = edge_index[0]
    dst = edge_index[1]

    bn = 1000
    m, gh = _pre_call(x, W, W_hh.T, b_hh.reshape(1, -1), bn)

    rows_per_tile = (-(-n // 16) + 7) // 8 * 8
    zeros = jnp.zeros((rows_per_tile, d), jnp.float32)

    nw, k, ngrp = 32, 128, 2
    e_pad = -(-e // (nw * ngrp * k)) * (nw * ngrp * k)
    gchunk = e_pad // (nw * ngrp * k)
    npad_rows = rows_per_tile * 16
    pad = e_pad - e
    if pad:
        pad_dst = n + jnp.arange(pad, dtype=jnp.int32) % (npad_rows - n)
        src_p = jnp.concatenate([src, jnp.zeros((pad,), jnp.int32)])
        dst_p = jnp.concatenate([dst, pad_dst])
    else:
        src_p, dst_p = src, dst
    src4 = src_p.reshape(nw, ngrp, gchunk, k)
    dst4 = dst_p.reshape(nw, ngrp, gchunk, k)
    part = _make_sc_seg_sum(n, d, e_pad, k, ngrp, gchunk)(m, src4, dst4, zeros)

    return _post_call(part[0, :n], part[1, :n], x, gh, W_ih.T,
                      b_ih.reshape(1, -1), bn)


# speedup vs baseline: 2.7534x; 2.7534x over previous
import functools

import jax
import jax.numpy as jnp
from jax import lax
from jax.experimental import pallas as pl
from jax.experimental.pallas import tpu as pltpu
from jax.experimental.pallas import tpu_sc as plsc


def _pre_body(x_ref, w_ref, whh_t_ref, bhh_ref, m_ref, gh_ref):
    xb = x_ref[...]
    m_ref[...] = jnp.dot(xb, w_ref[...], preferred_element_type=jnp.float32)
    gh_ref[...] = (
        jnp.dot(xb, whh_t_ref[...], preferred_element_type=jnp.float32)
        + bhh_ref[...]
    )


def _pre_call(x, w, whh_t, bhh, bn):
    n, d = x.shape
    d3 = whh_t.shape[1]
    grid = n // bn
    return pl.pallas_call(
        _pre_body,
        grid=(grid,),
        in_specs=[
            pl.BlockSpec((bn, d), lambda i: (i, 0)),
            pl.BlockSpec((d, d), lambda i: (0, 0)),
            pl.BlockSpec((d, d3), lambda i: (0, 0)),
            pl.BlockSpec((1, d3), lambda i: (0, 0)),
        ],
        out_specs=[
            pl.BlockSpec((bn, d), lambda i: (i, 0)),
            pl.BlockSpec((bn, d3), lambda i: (i, 0)),
        ],
        out_shape=[
            jax.ShapeDtypeStruct((n, d), jnp.float32),
            jax.ShapeDtypeStruct((n, d3), jnp.float32),
        ],
    )(x, w, whh_t, bhh)


def _make_sc_seg_sum(n, d, e_pad, k, ngrp, gchunk):
    info = plsc.get_sparse_core_info()
    nc, ns = info.num_cores, info.num_subcores
    nw = nc * ns
    assert e_pad == nw * ngrp * gchunk * k
    rows_per_tile = (-(-n // ns) + 7) // 8 * 8
    npad = rows_per_tile * ns

    mesh = plsc.VectorSubcoreMesh(core_axis_name="c", subcore_axis_name="s")

    @functools.partial(
        pl.kernel,
        out_type=jax.ShapeDtypeStruct((nc, npad, d), jnp.float32),
        mesh=mesh,
        scratch_types=[
            pltpu.VMEM((gchunk, k), jnp.int32),
            pltpu.VMEM((gchunk, k), jnp.int32),
            pltpu.VMEM((k, d), jnp.float32),
            pltpu.VMEM((k, d), jnp.float32),
            pltpu.VMEM_SHARED((npad, d), jnp.float32),
            pltpu.SemaphoreType.DMA,
            pltpu.SemaphoreType.DMA,
        ],
    )
    def seg_sum(m_hbm, src_hbm, dst_hbm, zeros_hbm, out_hbm,
                src_g, dst_g, buf_a, buf_b, agg_sh, sem_a, sem_b):
        cid = lax.axis_index("c")
        sid = lax.axis_index("s")
        wid = sid * nc + cid

        row0 = sid * rows_per_tile
        pltpu.sync_copy(zeros_hbm, agg_sh.at[pl.ds(row0, rows_per_tile)])
        plsc.subcore_barrier()

        for g in range(ngrp):
            pltpu.sync_copy(src_hbm.at[wid, g], src_g)
            pltpu.sync_copy(dst_hbm.at[wid, g], dst_g)

            pltpu.async_copy(m_hbm.at[src_g.at[0]], buf_a, sem_a)
            pltpu.async_copy(m_hbm.at[src_g.at[1]], buf_b, sem_b)

            def pair(j, carry):
                i = j * 2
                pltpu.make_async_copy(
                    m_hbm.at[src_g.at[i]], buf_a, sem_a).wait()
                pltpu.sync_copy(buf_a, agg_sh.at[dst_g.at[i]], add=True)

                @pl.when(i + 2 < gchunk)
                def _():
                    pltpu.async_copy(m_hbm.at[src_g.at[i + 2]], buf_a, sem_a)

                pltpu.make_async_copy(
                    m_hbm.at[src_g.at[i + 1]], buf_b, sem_b).wait()
                pltpu.sync_copy(buf_b, agg_sh.at[dst_g.at[i + 1]], add=True)

                @pl.when(i + 3 < gchunk)
                def _():
                    pltpu.async_copy(m_hbm.at[src_g.at[i + 3]], buf_b, sem_b)

                return carry

            lax.fori_loop(0, gchunk // 2, pair, 0)
            if gchunk % 2 == 1:
                last = gchunk - 1
                pltpu.make_async_copy(
                    m_hbm.at[src_g.at[last]], buf_a, sem_a).wait()
                pltpu.sync_copy(buf_a, agg_sh.at[dst_g.at[last]], add=True)

        plsc.subcore_barrier()
        pltpu.sync_copy(
            agg_sh.at[pl.ds(row0, rows_per_tile)],
            out_hbm.at[cid, pl.ds(row0, rows_per_tile)],
        )

    return seg_sum


def _post_body(p0_ref, p1_ref, x_ref, gh_ref, wih_t_ref, bih_ref, out_ref):
    d = x_ref.shape[1]
    agg = p0_ref[...] + p1_ref[...]
    gi = (
        jnp.dot(agg, wih_t_ref[...], preferred_element_type=jnp.float32)
        + bih_ref[...]
    )
    gh = gh_ref[...]
    xb = x_ref[...]
    r = jax.nn.sigmoid(gi[:, :d] + gh[:, :d])
    z = jax.nn.sigmoid(gi[:, d:2 * d] + gh[:, d:2 * d])
    nn = jnp.tanh(gi[:, 2 * d:] + r * gh[:, 2 * d:])
    h = (1.0 - z) * nn + z * xb
    out_ref[...] = xb + jnp.maximum(h, 0.0)


def _post_call(p0, p1, x, gh, wih_t, bih, bn):
    n, d = x.shape
    d3 = wih_t.shape[1]
    grid = n // bn
    return pl.pallas_call(
        _post_body,
        grid=(grid,),
        in_specs=[
            pl.BlockSpec((bn, d), lambda i: (i, 0)),
            pl.BlockSpec((bn, d), lambda i: (i, 0)),
            pl.BlockSpec((bn, d), lambda i: (i, 0)),
            pl.BlockSpec((bn, d3), lambda i: (i, 0)),
            pl.BlockSpec((d, d3), lambda i: (0, 0)),
            pl.BlockSpec((1, d3), lambda i: (0, 0)),
        ],
        out_specs=pl.BlockSpec((bn, d), lambda i: (i, 0)),
        out_shape=jax.ShapeDtypeStruct((n, d), jnp.float32),
    )(p0, p1, x, gh, wih_t, bih)


def kernel(x, edge_index, W, W_ih, W_hh, b_ih, b_hh):
    n, d = x.shape
    e = edge_index.shape[1]
    src = edge_index[0]
    dst = edge_index[1]

    bn = 1000
    m, gh = _pre_call(x, W, W_hh.T, b_hh.reshape(1, -1), bn)

    rows_per_tile = (-(-n // 16) + 7) // 8 * 8
    zeros = jnp.zeros((rows_per_tile, d), jnp.float32)

    nw, k, ngrp = 32, 80, 5
    e_pad = -(-e // (nw * ngrp * k)) * (nw * ngrp * k)
    gchunk = e_pad // (nw * ngrp * k)
    npad_rows = rows_per_tile * 16
    pad = e_pad - e
    if pad:
        pad_dst = n + jnp.arange(pad, dtype=jnp.int32) % (npad_rows - n)
        src_p = jnp.concatenate([src, jnp.zeros((pad,), jnp.int32)])
        dst_p = jnp.concatenate([dst, pad_dst])
    else:
        src_p, dst_p = src, dst
    src4 = src_p.reshape(nw, ngrp, gchunk, k)
    dst4 = dst_p.reshape(nw, ngrp, gchunk, k)
    part = _make_sc_seg_sum(n, d, e_pad, k, ngrp, gchunk)(m, src4, dst4, zeros)

    return _post_call(part[0, :n], part[1, :n], x, gh, W_ih.T,
                      b_ih.reshape(1, -1), bn)


# speedup vs baseline: 3.1918x; 1.1592x over previous
import functools

import jax
import jax.numpy as jnp
from jax import lax
from jax.experimental import pallas as pl
from jax.experimental.pallas import tpu as pltpu
from jax.experimental.pallas import tpu_sc as plsc


def _pre_body(x_ref, w_ref, whh_ref, bhh_ref, m_ref, gh_ref):
    xb = x_ref[...]
    m_ref[...] = jnp.dot(xb, w_ref[...], preferred_element_type=jnp.float32)
    gh_ref[...] = (
        lax.dot_general(xb, whh_ref[...], (((1,), (1,)), ((), ())),
                        preferred_element_type=jnp.float32)
        + bhh_ref[...]
    )


def _pre_call(x, w, whh, bhh, bn):
    n, d = x.shape
    d3 = whh.shape[0]
    grid = n // bn
    return pl.pallas_call(
        _pre_body,
        grid=(grid,),
        in_specs=[
            pl.BlockSpec((bn, d), lambda i: (i, 0)),
            pl.BlockSpec((d, d), lambda i: (0, 0)),
            pl.BlockSpec((d3, d), lambda i: (0, 0)),
            pl.BlockSpec((1, d3), lambda i: (0, 0)),
        ],
        out_specs=[
            pl.BlockSpec((bn, d), lambda i: (i, 0)),
            pl.BlockSpec((bn, d3), lambda i: (i, 0)),
        ],
        out_shape=[
            jax.ShapeDtypeStruct((n, d), jnp.float32),
            jax.ShapeDtypeStruct((n, d3), jnp.float32),
        ],
    )(x, w, whh, bhh)


def _make_sc_seg_sum(n, d, e_pad, k, ngrp, gchunk):
    info = plsc.get_sparse_core_info()
    nc, ns = info.num_cores, info.num_subcores
    nw = nc * ns
    assert e_pad == nw * ngrp * gchunk * k
    rows_per_tile = (-(-n // ns) + 7) // 8 * 8
    npad = rows_per_tile * ns

    mesh = plsc.VectorSubcoreMesh(core_axis_name="c", subcore_axis_name="s")

    @functools.partial(
        pl.kernel,
        out_type=jax.ShapeDtypeStruct((nc, npad, d), jnp.float32),
        mesh=mesh,
        scratch_types=[
            pltpu.VMEM((gchunk, k), jnp.int32),
            pltpu.VMEM((gchunk, k), jnp.int32),
            pltpu.VMEM((k, d), jnp.float32),
            pltpu.VMEM((k, d), jnp.float32),
            pltpu.VMEM((k, d), jnp.float32),
            pltpu.VMEM_SHARED((npad, d), jnp.float32),
            pltpu.SemaphoreType.DMA,
            pltpu.SemaphoreType.DMA,
            pltpu.SemaphoreType.DMA,
            pltpu.SemaphoreType.DMA,
            pltpu.SemaphoreType.DMA,
            pltpu.SemaphoreType.DMA,
        ],
    )
    def seg_sum(m_hbm, src_hbm, dst_hbm, zeros_hbm, out_hbm,
                src_g, dst_g, buf0, buf1, buf2, agg_sh,
                semg0, semg1, semg2, sems0, sems1, sems2):
        cid = lax.axis_index("c")
        sid = lax.axis_index("s")
        wid = sid * nc + cid
        bufs = (buf0, buf1, buf2)
        semg = (semg0, semg1, semg2)
        sems = (sems0, sems1, sems2)

        def g_start(c, b):
            pltpu.async_copy(m_hbm.at[src_g.at[c]], bufs[b], semg[b])

        def g_wait(c, b):
            pltpu.make_async_copy(m_hbm.at[src_g.at[c]], bufs[b],
                                  semg[b]).wait()

        def s_start(c, b):
            pltpu.async_copy(bufs[b], agg_sh.at[dst_g.at[c]], sems[b],
                             add=True)

        def s_wait(c, b):
            pltpu.make_async_copy(bufs[b], agg_sh.at[dst_g.at[c]],
                                  sems[b]).wait()

        row0 = sid * rows_per_tile
        pltpu.sync_copy(zeros_hbm, agg_sh.at[pl.ds(row0, rows_per_tile)])
        plsc.subcore_barrier()

        for g in range(ngrp):
            pltpu.sync_copy(src_hbm.at[wid, g], src_g)
            pltpu.sync_copy(dst_hbm.at[wid, g], dst_g)

            g_start(0, 0)
            g_start(1, 1)
            g_wait(0, 0)
            s_start(0, 0)
            g_start(2, 2)

            def slot(c, b):
                bp = (b + 2) % 3
                g_wait(c, b)
                s_wait(c - 1, bp)
                s_start(c, b)

                @pl.when(c + 2 < gchunk)
                def _():
                    g_start(c + 2, bp)

            def triple(t, carry):
                c0 = 1 + t * 3
                for o in range(3):
                    slot(c0 + o, (1 + o) % 3)
                return carry

            n_triples = (gchunk - 1) // 3
            lax.fori_loop(0, n_triples, triple, 0)
            for c in range(1 + 3 * n_triples, gchunk):
                slot(c, c % 3)
            s_wait(gchunk - 1, (gchunk - 1) % 3)

        plsc.subcore_barrier()
        pltpu.sync_copy(
            agg_sh.at[pl.ds(row0, rows_per_tile)],
            out_hbm.at[cid, pl.ds(row0, rows_per_tile)],
        )

    return seg_sum


def _post_body(p0_ref, p1_ref, x_ref, gh_ref, wih_ref, bih_ref, out_ref):
    d = x_ref.shape[1]
    agg = p0_ref[0] + p1_ref[0]
    gi = (
        lax.dot_general(agg, wih_ref[...], (((1,), (1,)), ((), ())),
                        preferred_element_type=jnp.float32)
        + bih_ref[...]
    )
    gh = gh_ref[...]
    xb = x_ref[...]
    r = jax.nn.sigmoid(gi[:, :d] + gh[:, :d])
    z = jax.nn.sigmoid(gi[:, d:2 * d] + gh[:, d:2 * d])
    nn = jnp.tanh(gi[:, 2 * d:] + r * gh[:, 2 * d:])
    h = (1.0 - z) * nn + z * xb
    out_ref[...] = xb + jnp.maximum(h, 0.0)


def _post_call(part, x, gh, wih, bih, bn):
    n, d = x.shape
    d3 = wih.shape[0]
    grid = n // bn
    return pl.pallas_call(
        _post_body,
        grid=(grid,),
        in_specs=[
            pl.BlockSpec((1, bn, d), lambda i: (0, i, 0)),
            pl.BlockSpec((1, bn, d), lambda i: (1, i, 0)),
            pl.BlockSpec((bn, d), lambda i: (i, 0)),
            pl.BlockSpec((bn, d3), lambda i: (i, 0)),
            pl.BlockSpec((d3, d), lambda i: (0, 0)),
            pl.BlockSpec((1, d3), lambda i: (0, 0)),
        ],
        out_specs=pl.BlockSpec((bn, d), lambda i: (i, 0)),
        out_shape=jax.ShapeDtypeStruct((n, d), jnp.float32),
    )(part, part, x, gh, wih, bih)


def kernel(x, edge_index, W, W_ih, W_hh, b_ih, b_hh):
    n, d = x.shape
    e = edge_index.shape[1]
    src = edge_index[0]
    dst = edge_index[1]

    bn = 1000
    m, gh = _pre_call(x, W, W_hh, b_hh.reshape(1, -1), bn)

    rows_per_tile = (-(-n // 16) + 7) // 8 * 8
    zeros = jnp.zeros((rows_per_tile, d), jnp.float32)

    nw, k, ngrp = 32, 80, 5
    e_pad = -(-e // (nw * ngrp * k)) * (nw * ngrp * k)
    gchunk = e_pad // (nw * ngrp * k)
    npad_rows = rows_per_tile * 16
    pad = e_pad - e
    if pad:
        pad_dst = n + jnp.arange(pad, dtype=jnp.int32) % (npad_rows - n)
        src_p = jnp.concatenate([src, jnp.zeros((pad,), jnp.int32)])
        dst_p = jnp.concatenate([dst, pad_dst])
    else:
        src_p, dst_p = src, dst
    src4 = src_p.reshape(nw, ngrp, gchunk, k)
    dst4 = dst_p.reshape(nw, ngrp, gchunk, k)
    part = _make_sc_seg_sum(n, d, e_pad, k, ngrp, gchunk)(m, src4, dst4, zeros)

    return _post_call(part, x, gh, W_ih, b_ih.reshape(1, -1), bn)


# speedup vs baseline: 3.2826x; 1.0284x over previous
import functools

import jax
import jax.numpy as jnp
from jax import lax
from jax.experimental import pallas as pl
from jax.experimental.pallas import tpu as pltpu
from jax.experimental.pallas import tpu_sc as plsc


def _pre_body(x_ref, w_ref, whh_ref, bhh_ref, m_ref, gh_ref):
    xb = x_ref[...]
    m_ref[...] = jnp.dot(xb, w_ref[...], preferred_element_type=jnp.float32)
    gh_ref[...] = (
        lax.dot_general(xb, whh_ref[...], (((1,), (1,)), ((), ())),
                        preferred_element_type=jnp.float32)
        + bhh_ref[...]
    )


def _pre_call(x, w, whh, bhh, bn):
    n, d = x.shape
    d3 = whh.shape[0]
    grid = n // bn
    return pl.pallas_call(
        _pre_body,
        grid=(grid,),
        in_specs=[
            pl.BlockSpec((bn, d), lambda i: (i, 0)),
            pl.BlockSpec((d, d), lambda i: (0, 0)),
            pl.BlockSpec((d3, d), lambda i: (0, 0)),
            pl.BlockSpec((1, d3), lambda i: (0, 0)),
        ],
        out_specs=[
            pl.BlockSpec((bn, d), lambda i: (i, 0)),
            pl.BlockSpec((bn, d3), lambda i: (i, 0)),
        ],
        out_shape=[
            jax.ShapeDtypeStruct((n, d), jnp.float32),
            jax.ShapeDtypeStruct((n, d3), jnp.float32),
        ],
    )(x, w, whh, bhh)


def _make_sc_seg_sum(n, d, e_pad, k, ngrp, gchunk):
    info = plsc.get_sparse_core_info()
    nc, ns = info.num_cores, info.num_subcores
    nw = nc * ns
    assert e_pad == nw * ngrp * gchunk * k
    rows_per_tile = (-(-n // ns) + 7) // 8 * 8
    npad = rows_per_tile * ns

    mesh = plsc.VectorSubcoreMesh(core_axis_name="c", subcore_axis_name="s")

    @functools.partial(
        pl.kernel,
        out_type=jax.ShapeDtypeStruct((nc, npad, d), jnp.float32),
        mesh=mesh,
        scratch_types=[
            pltpu.VMEM((gchunk, k), jnp.int32),
            pltpu.VMEM((gchunk, k), jnp.int32),
            pltpu.VMEM((k, d), jnp.float32),
            pltpu.VMEM((k, d), jnp.float32),
            pltpu.VMEM((k, d), jnp.float32),
            pltpu.VMEM((k, d), jnp.float32),
            pltpu.VMEM_SHARED((npad, d), jnp.float32),
            pltpu.SemaphoreType.DMA,
            pltpu.SemaphoreType.DMA,
            pltpu.SemaphoreType.DMA,
            pltpu.SemaphoreType.DMA,
            pltpu.SemaphoreType.DMA,
            pltpu.SemaphoreType.DMA,
            pltpu.SemaphoreType.DMA,
            pltpu.SemaphoreType.DMA,
        ],
    )
    def seg_sum(m_hbm, src_hbm, dst_hbm, zeros_hbm, out_hbm,
                src_g, dst_g, buf0, buf1, buf2, buf3, agg_sh,
                semg0, semg1, semg2, semg3, sems0, sems1, sems2, sems3):
        cid = lax.axis_index("c")
        sid = lax.axis_index("s")
        wid = sid * nc + cid
        bufs = (buf0, buf1, buf2, buf3)
        semg = (semg0, semg1, semg2, semg3)
        sems = (sems0, sems1, sems2, sems3)

        def g_start(c, b):
            pltpu.async_copy(m_hbm.at[src_g.at[c]], bufs[b], semg[b])

        def g_wait(c, b):
            pltpu.make_async_copy(m_hbm.at[src_g.at[c]], bufs[b],
                                  semg[b]).wait()

        def s_start(c, b):
            pltpu.async_copy(bufs[b], agg_sh.at[dst_g.at[c]], sems[b],
                             add=True)

        def s_wait(c, b):
            pltpu.make_async_copy(bufs[b], agg_sh.at[dst_g.at[c]],
                                  sems[b]).wait()

        row0 = sid * rows_per_tile
        pltpu.sync_copy(zeros_hbm, agg_sh.at[pl.ds(row0, rows_per_tile)])
        plsc.subcore_barrier()

        for g in range(ngrp):
            pltpu.sync_copy(src_hbm.at[wid, g], src_g)
            pltpu.sync_copy(dst_hbm.at[wid, g], dst_g)

            g_start(0, 0)
            g_start(1, 1)
            g_start(2, 2)
            g_wait(0, 0)
            s_start(0, 0)
            g_start(3, 3)

            def slot(c, b):
                bf = (b + 3) % 4
                g_wait(c, b)
                s_wait(c - 1, bf)
                s_start(c, b)

                @pl.when(c + 3 < gchunk)
                def _():
                    g_start(c + 3, bf)

            def quad(t, carry):
                c0 = 1 + t * 4
                for o in range(4):
                    slot(c0 + o, (1 + o) % 4)
                return carry

            n_quads = (gchunk - 1) // 4
            lax.fori_loop(0, n_quads, quad, 0)
            for c in range(1 + 4 * n_quads, gchunk):
                slot(c, c % 4)
            s_wait(gchunk - 1, (gchunk - 1) % 4)

        plsc.subcore_barrier()
        pltpu.sync_copy(
            agg_sh.at[pl.ds(row0, rows_per_tile)],
            out_hbm.at[cid, pl.ds(row0, rows_per_tile)],
        )

    return seg_sum


def _post_body(p0_ref, p1_ref, x_ref, gh_ref, wih_ref, bih_ref, out_ref):
    d = x_ref.shape[1]
    agg = p0_ref[0] + p1_ref[0]
    gi = (
        lax.dot_general(agg, wih_ref[...], (((1,), (1,)), ((), ())),
                        preferred_element_type=jnp.float32)
        + bih_ref[...]
    )
    gh = gh_ref[...]
    xb = x_ref[...]
    r = jax.nn.sigmoid(gi[:, :d] + gh[:, :d])
    z = jax.nn.sigmoid(gi[:, d:2 * d] + gh[:, d:2 * d])
    nn = jnp.tanh(gi[:, 2 * d:] + r * gh[:, 2 * d:])
    h = (1.0 - z) * nn + z * xb
    out_ref[...] = xb + jnp.maximum(h, 0.0)


def _post_call(part, x, gh, wih, bih, bn):
    n, d = x.shape
    d3 = wih.shape[0]
    grid = n // bn
    return pl.pallas_call(
        _post_body,
        grid=(grid,),
        in_specs=[
            pl.BlockSpec((1, bn, d), lambda i: (0, i, 0)),
            pl.BlockSpec((1, bn, d), lambda i: (1, i, 0)),
            pl.BlockSpec((bn, d), lambda i: (i, 0)),
            pl.BlockSpec((bn, d3), lambda i: (i, 0)),
            pl.BlockSpec((d3, d), lambda i: (0, 0)),
            pl.BlockSpec((1, d3), lambda i: (0, 0)),
        ],
        out_specs=pl.BlockSpec((bn, d), lambda i: (i, 0)),
        out_shape=jax.ShapeDtypeStruct((n, d), jnp.float32),
    )(part, part, x, gh, wih, bih)


def kernel(x, edge_index, W, W_ih, W_hh, b_ih, b_hh):
    n, d = x.shape
    e = edge_index.shape[1]
    src = edge_index[0]
    dst = edge_index[1]

    bn = 1000
    m, gh = _pre_call(x, W, W_hh, b_hh.reshape(1, -1), bn)

    rows_per_tile = (-(-n // 16) + 7) // 8 * 8
    zeros = jnp.zeros((rows_per_tile, d), jnp.float32)

    nw, k, ngrp = 32, 80, 5
    e_pad = -(-e // (nw * ngrp * k)) * (nw * ngrp * k)
    gchunk = e_pad // (nw * ngrp * k)
    npad_rows = rows_per_tile * 16
    pad = e_pad - e
    if pad:
        pad_dst = n + jnp.arange(pad, dtype=jnp.int32) % (npad_rows - n)
        src_p = jnp.concatenate([src, jnp.zeros((pad,), jnp.int32)])
        dst_p = jnp.concatenate([dst, pad_dst])
    else:
        src_p, dst_p = src, dst
    src4 = src_p.reshape(nw, ngrp, gchunk, k)
    dst4 = dst_p.reshape(nw, ngrp, gchunk, k)
    part = _make_sc_seg_sum(n, d, e_pad, k, ngrp, gchunk)(m, src4, dst4, zeros)

    return _post_call(part, x, gh, W_ih, b_ih.reshape(1, -1), bn)


# speedup vs baseline: 3.3960x; 1.0346x over previous
import functools

import jax
import jax.numpy as jnp
from jax import lax
from jax.experimental import pallas as pl
from jax.experimental.pallas import tpu as pltpu
from jax.experimental.pallas import tpu_sc as plsc


def _pre_body(x_ref, w_ref, m_ref):
    m_ref[...] = jnp.dot(x_ref[...], w_ref[...],
                         preferred_element_type=jnp.float32)


def _pre_call(x, w, bn):
    n, d = x.shape
    grid = n // bn
    return pl.pallas_call(
        _pre_body,
        grid=(grid,),
        in_specs=[
            pl.BlockSpec((bn, d), lambda i: (i, 0)),
            pl.BlockSpec((d, d), lambda i: (0, 0)),
        ],
        out_specs=pl.BlockSpec((bn, d), lambda i: (i, 0)),
        out_shape=jax.ShapeDtypeStruct((n, d), jnp.float32),
    )(x, w)


def _make_sc_seg_sum(n, d, e_pad, k, ngrp, gchunk):
    info = plsc.get_sparse_core_info()
    nc, ns = info.num_cores, info.num_subcores
    nw = nc * ns
    assert e_pad == nw * ngrp * gchunk * k
    rows_per_tile = (-(-n // ns) + 7) // 8 * 8
    npad = rows_per_tile * ns

    mesh = plsc.VectorSubcoreMesh(core_axis_name="c", subcore_axis_name="s")

    @functools.partial(
        pl.kernel,
        out_type=jax.ShapeDtypeStruct((nc, npad, d), jnp.float32),
        mesh=mesh,
        scratch_types=[
            pltpu.VMEM((gchunk, k), jnp.int32),
            pltpu.VMEM((gchunk, k), jnp.int32),
            pltpu.VMEM((k, d), jnp.float32),
            pltpu.VMEM((k, d), jnp.float32),
            pltpu.VMEM((k, d), jnp.float32),
            pltpu.VMEM((k, d), jnp.float32),
            pltpu.VMEM_SHARED((npad, d), jnp.float32),
            pltpu.SemaphoreType.DMA,
            pltpu.SemaphoreType.DMA,
            pltpu.SemaphoreType.DMA,
            pltpu.SemaphoreType.DMA,
            pltpu.SemaphoreType.DMA,
            pltpu.SemaphoreType.DMA,
            pltpu.SemaphoreType.DMA,
            pltpu.SemaphoreType.DMA,
        ],
    )
    def seg_sum(m_hbm, src_hbm, dst_hbm, zeros_hbm, out_hbm,
                src_g, dst_g, buf0, buf1, buf2, buf3, agg_sh,
                semg0, semg1, semg2, semg3, sems0, sems1, sems2, sems3):
        cid = lax.axis_index("c")
        sid = lax.axis_index("s")
        wid = sid * nc + cid
        bufs = (buf0, buf1, buf2, buf3)
        semg = (semg0, semg1, semg2, semg3)
        sems = (sems0, sems1, sems2, sems3)

        def g_start(c, b):
            pltpu.async_copy(m_hbm.at[src_g.at[c]], bufs[b], semg[b])

        def g_wait(c, b):
            pltpu.make_async_copy(m_hbm.at[src_g.at[c]], bufs[b],
                                  semg[b]).wait()

        def s_start(c, b):
            pltpu.async_copy(bufs[b], agg_sh.at[dst_g.at[c]], sems[b],
                             add=True)

        def s_wait(c, b):
            pltpu.make_async_copy(bufs[b], agg_sh.at[dst_g.at[c]],
                                  sems[b]).wait()

        row0 = sid * rows_per_tile
        pltpu.sync_copy(zeros_hbm, agg_sh.at[pl.ds(row0, rows_per_tile)])
        plsc.subcore_barrier()

        for g in range(ngrp):
            pltpu.sync_copy(src_hbm.at[wid, g], src_g)
            pltpu.sync_copy(dst_hbm.at[wid, g], dst_g)

            g_start(0, 0)
            g_start(1, 1)
            g_start(2, 2)
            g_wait(0, 0)
            s_start(0, 0)
            g_start(3, 3)

            def slot(c, b):
                bf = (b + 3) % 4
                g_wait(c, b)
                s_wait(c - 1, bf)
                s_start(c, b)

                @pl.when(c + 3 < gchunk)
                def _():
                    g_start(c + 3, bf)

            def quad(t, carry):
                c0 = 1 + t * 4
                for o in range(4):
                    slot(c0 + o, (1 + o) % 4)
                return carry

            n_quads = (gchunk - 1) // 4
            lax.fori_loop(0, n_quads, quad, 0)
            for c in range(1 + 4 * n_quads, gchunk):
                slot(c, c % 4)
            s_wait(gchunk - 1, (gchunk - 1) % 4)

        plsc.subcore_barrier()
        pltpu.sync_copy(
            agg_sh.at[pl.ds(row0, rows_per_tile)],
            out_hbm.at[cid, pl.ds(row0, rows_per_tile)],
        )

    return seg_sum


def _post_body(p0_ref, p1_ref, x_ref, wih_ref, whh_ref, bih_ref, bhh_ref,
               out_ref):
    d = x_ref.shape[1]
    xb = x_ref[...]
    agg = p0_ref[0] + p1_ref[0]
    gi = (
        lax.dot_general(agg, wih_ref[...], (((1,), (1,)), ((), ())),
                        preferred_element_type=jnp.float32)
        + bih_ref[...]
    )
    gh = (
        lax.dot_general(xb, whh_ref[...], (((1,), (1,)), ((), ())),
                        preferred_element_type=jnp.float32)
        + bhh_ref[...]
    )
    r = jax.nn.sigmoid(gi[:, :d] + gh[:, :d])
    z = jax.nn.sigmoid(gi[:, d:2 * d] + gh[:, d:2 * d])
    nn = jnp.tanh(gi[:, 2 * d:] + r * gh[:, 2 * d:])
    h = (1.0 - z) * nn + z * xb
    out_ref[...] = xb + jnp.maximum(h, 0.0)


def _post_call(part, x, wih, whh, bih, bhh, bn):
    n, d = x.shape
    d3 = wih.shape[0]
    grid = n // bn
    return pl.pallas_call(
        _post_body,
        grid=(grid,),
        in_specs=[
            pl.BlockSpec((1, bn, d), lambda i: (0, i, 0)),
            pl.BlockSpec((1, bn, d), lambda i: (1, i, 0)),
            pl.BlockSpec((bn, d), lambda i: (i, 0)),
            pl.BlockSpec((d3, d), lambda i: (0, 0)),
            pl.BlockSpec((d3, d), lambda i: (0, 0)),
            pl.BlockSpec((1, d3), lambda i: (0, 0)),
            pl.BlockSpec((1, d3), lambda i: (0, 0)),
        ],
        out_specs=pl.BlockSpec((bn, d), lambda i: (i, 0)),
        out_shape=jax.ShapeDtypeStruct((n, d), jnp.float32),
    )(part, part, x, wih, whh, bih, bhh)


def kernel(x, edge_index, W, W_ih, W_hh, b_ih, b_hh):
    n, d = x.shape
    e = edge_index.shape[1]
    src = edge_index[0]
    dst = edge_index[1]

    bn = 1000
    m = _pre_call(x, W, bn)

    rows_per_tile = (-(-n // 16) + 7) // 8 * 8
    zeros = jnp.zeros((rows_per_tile, d), jnp.float32)

    nw, k, ngrp = 32, 80, 5
    e_pad = -(-e // (nw * ngrp * k)) * (nw * ngrp * k)
    gchunk = e_pad // (nw * ngrp * k)
    npad_rows = rows_per_tile * 16
    pad = e_pad - e
    if pad:
        pad_dst = n + jnp.arange(pad, dtype=jnp.int32) % (npad_rows - n)
        src_p = jnp.concatenate([src, jnp.zeros((pad,), jnp.int32)])
        dst_p = jnp.concatenate([dst, pad_dst])
    else:
        src_p, dst_p = src, dst
    src4 = src_p.reshape(nw, ngrp, gchunk, k)
    dst4 = dst_p.reshape(nw, ngrp, gchunk, k)
    part = _make_sc_seg_sum(n, d, e_pad, k, ngrp, gchunk)(m, src4, dst4, zeros)

    return _post_call(part, x, W_ih, W_hh, b_ih.reshape(1, -1),
                      b_hh.reshape(1, -1), bn)


# speedup vs baseline: 3.5873x; 1.0563x over previous
import functools

import jax
import jax.numpy as jnp
from jax import lax
from jax.experimental import pallas as pl
from jax.experimental.pallas import tpu as pltpu
from jax.experimental.pallas import tpu_sc as plsc


def _make_sc_seg_sum(n, d, e_pad, k, ngrp, gchunk):
    info = plsc.get_sparse_core_info()
    nc, ns = info.num_cores, info.num_subcores
    nw = nc * ns
    assert e_pad == nw * ngrp * gchunk * k
    rows_per_tile = (-(-n // ns) + 7) // 8 * 8
    npad = rows_per_tile * ns

    mesh = plsc.VectorSubcoreMesh(core_axis_name="c", subcore_axis_name="s")

    @functools.partial(
        pl.kernel,
        out_type=jax.ShapeDtypeStruct((nc, npad, d), jnp.float32),
        mesh=mesh,
        scratch_types=[
            pltpu.VMEM((gchunk, k), jnp.int32),
            pltpu.VMEM((gchunk, k), jnp.int32),
            pltpu.VMEM((k, d), jnp.float32),
            pltpu.VMEM((k, d), jnp.float32),
            pltpu.VMEM((k, d), jnp.float32),
            pltpu.VMEM((k, d), jnp.float32),
            pltpu.VMEM_SHARED((npad, d), jnp.float32),
            pltpu.SemaphoreType.DMA,
            pltpu.SemaphoreType.DMA,
            pltpu.SemaphoreType.DMA,
            pltpu.SemaphoreType.DMA,
            pltpu.SemaphoreType.DMA,
            pltpu.SemaphoreType.DMA,
            pltpu.SemaphoreType.DMA,
            pltpu.SemaphoreType.DMA,
        ],
    )
    def seg_sum(m_hbm, src_hbm, dst_hbm, zeros_hbm, out_hbm,
                src_g, dst_g, buf0, buf1, buf2, buf3, agg_sh,
                semg0, semg1, semg2, semg3, sems0, sems1, sems2, sems3):
        cid = lax.axis_index("c")
        sid = lax.axis_index("s")
        wid = sid * nc + cid
        bufs = (buf0, buf1, buf2, buf3)
        semg = (semg0, semg1, semg2, semg3)
        sems = (sems0, sems1, sems2, sems3)

        def g_start(c, b):
            pltpu.async_copy(m_hbm.at[src_g.at[c]], bufs[b], semg[b])

        def g_wait(c, b):
            pltpu.make_async_copy(m_hbm.at[src_g.at[c]], bufs[b],
                                  semg[b]).wait()

        def s_start(c, b):
            pltpu.async_copy(bufs[b], agg_sh.at[dst_g.at[c]], sems[b],
                             add=True)

        def s_wait(c, b):
            pltpu.make_async_copy(bufs[b], agg_sh.at[dst_g.at[c]],
                                  sems[b]).wait()

        row0 = sid * rows_per_tile
        pltpu.sync_copy(zeros_hbm, agg_sh.at[pl.ds(row0, rows_per_tile)])
        plsc.subcore_barrier()

        for g in range(ngrp):
            pltpu.sync_copy(src_hbm.at[wid, g], src_g)
            pltpu.sync_copy(dst_hbm.at[wid, g], dst_g)

            g_start(0, 0)
            g_start(1, 1)
            g_start(2, 2)
            g_wait(0, 0)
            s_start(0, 0)
            g_start(3, 3)

            def slot(c, b):
                bf = (b + 3) % 4
                g_wait(c, b)
                s_wait(c - 1, bf)
                s_start(c, b)

                @pl.when(c + 3 < gchunk)
                def _():
                    g_start(c + 3, bf)

            def quad(t, carry):
                c0 = 1 + t * 4
                for o in range(4):
                    slot(c0 + o, (1 + o) % 4)
                return carry

            n_quads = (gchunk - 1) // 4
            lax.fori_loop(0, n_quads, quad, 0)
            for c in range(1 + 4 * n_quads, gchunk):
                slot(c, c % 4)
            s_wait(gchunk - 1, (gchunk - 1) % 4)

        plsc.subcore_barrier()
        pltpu.sync_copy(
            agg_sh.at[pl.ds(row0, rows_per_tile)],
            out_hbm.at[cid, pl.ds(row0, rows_per_tile)],
        )

    return seg_sum


def _post_body(p0_ref, p1_ref, x_ref, w_ref, wih_ref, whh_ref, bih_ref,
               bhh_ref, out_ref):
    d = x_ref.shape[1]
    xb = x_ref[...]
    w2 = lax.dot_general(w_ref[...], wih_ref[...], (((1,), (1,)), ((), ())),
                         preferred_element_type=jnp.float32)
    sx = p0_ref[0] + p1_ref[0]
    gi = (
        jnp.dot(sx, w2, preferred_element_type=jnp.float32)
        + bih_ref[...]
    )
    gh = (
        lax.dot_general(xb, whh_ref[...], (((1,), (1,)), ((), ())),
                        preferred_element_type=jnp.float32)
        + bhh_ref[...]
    )
    r = jax.nn.sigmoid(gi[:, :d] + gh[:, :d])
    z = jax.nn.sigmoid(gi[:, d:2 * d] + gh[:, d:2 * d])
    nn = jnp.tanh(gi[:, 2 * d:] + r * gh[:, 2 * d:])
    h = (1.0 - z) * nn + z * xb
    out_ref[...] = xb + jnp.maximum(h, 0.0)


def _post_call(part, x, w, wih, whh, bih, bhh, bn):
    n, d = x.shape
    d3 = wih.shape[0]
    grid = n // bn
    return pl.pallas_call(
        _post_body,
        grid=(grid,),
        in_specs=[
            pl.BlockSpec((1, bn, d), lambda i: (0, i, 0)),
            pl.BlockSpec((1, bn, d), lambda i: (1, i, 0)),
            pl.BlockSpec((bn, d), lambda i: (i, 0)),
            pl.BlockSpec((d, d), lambda i: (0, 0)),
            pl.BlockSpec((d3, d), lambda i: (0, 0)),
            pl.BlockSpec((d3, d), lambda i: (0, 0)),
            pl.BlockSpec((1, d3), lambda i: (0, 0)),
            pl.BlockSpec((1, d3), lambda i: (0, 0)),
        ],
        out_specs=pl.BlockSpec((bn, d), lambda i: (i, 0)),
        out_shape=jax.ShapeDtypeStruct((n, d), jnp.float32),
    )(part, part, x, w, wih, whh, bih, bhh)


def kernel(x, edge_index, W, W_ih, W_hh, b_ih, b_hh):
    n, d = x.shape
    e = edge_index.shape[1]
    src = edge_index[0]
    dst = edge_index[1]

    bn = 1000
    rows_per_tile = (-(-n // 16) + 7) // 8 * 8
    zeros = jnp.zeros((rows_per_tile, d), jnp.float32)

    nw, k, ngrp = 32, 80, 5
    e_pad = -(-e // (nw * ngrp * k)) * (nw * ngrp * k)
    gchunk = e_pad // (nw * ngrp * k)
    npad_rows = rows_per_tile * 16
    pad = e_pad - e
    if pad:
        pad_dst = n + jnp.arange(pad, dtype=jnp.int32) % (npad_rows - n)
        src_p = jnp.concatenate([src, jnp.zeros((pad,), jnp.int32)])
        dst_p = jnp.concatenate([dst, pad_dst])
    else:
        src_p, dst_p = src, dst
    src4 = src_p.reshape(nw, ngrp, gchunk, k)
    dst4 = dst_p.reshape(nw, ngrp, gchunk, k)
    part = _make_sc_seg_sum(n, d, e_pad, k, ngrp, gchunk)(x, src4, dst4, zeros)

    return _post_call(part, x, W, W_ih, W_hh, b_ih.reshape(1, -1),
                      b_hh.reshape(1, -1), bn)


# speedup vs baseline: 3.6612x; 1.0206x over previous
import functools

import jax
import jax.numpy as jnp
from jax import lax
from jax.experimental import pallas as pl
from jax.experimental.pallas import tpu as pltpu
from jax.experimental.pallas import tpu_sc as plsc


def _make_sc_seg_sum(n, d, e_pad, k, ngrp, gchunk):
    info = plsc.get_sparse_core_info()
    nc, ns = info.num_cores, info.num_subcores
    nw = nc * ns
    assert e_pad == nw * ngrp * gchunk * k
    rows_per_tile = (-(-n // ns) + 7) // 8 * 8
    npad = rows_per_tile * ns

    mesh = plsc.VectorSubcoreMesh(core_axis_name="c", subcore_axis_name="s")

    @functools.partial(
        pl.kernel,
        out_type=jax.ShapeDtypeStruct((nc, npad, d), jnp.float32),
        mesh=mesh,
        scratch_types=[
            pltpu.VMEM((gchunk, k), jnp.int32),
            pltpu.VMEM((gchunk, k), jnp.int32),
            pltpu.VMEM((k, d), jnp.float32),
            pltpu.VMEM((k, d), jnp.float32),
            pltpu.VMEM((k, d), jnp.float32),
            pltpu.VMEM((k, d), jnp.float32),
            pltpu.VMEM_SHARED((npad, d), jnp.float32),
            pltpu.SemaphoreType.DMA,
            pltpu.SemaphoreType.DMA,
            pltpu.SemaphoreType.DMA,
            pltpu.SemaphoreType.DMA,
            pltpu.SemaphoreType.DMA,
            pltpu.SemaphoreType.DMA,
            pltpu.SemaphoreType.DMA,
            pltpu.SemaphoreType.DMA,
        ],
    )
    def seg_sum(m_hbm, src_hbm, dst_hbm, zeros_hbm, out_hbm,
                src_g, dst_g, buf0, buf1, buf2, buf3, agg_sh,
                semg0, semg1, semg2, semg3, sems0, sems1, sems2, sems3):
        cid = lax.axis_index("c")
        sid = lax.axis_index("s")
        wid = sid * nc + cid
        bufs = (buf0, buf1, buf2, buf3)
        semg = (semg0, semg1, semg2, semg3)
        sems = (sems0, sems1, sems2, sems3)

        def g_start(c, b):
            pltpu.async_copy(m_hbm.at[src_g.at[c]], bufs[b], semg[b])

        def g_wait(c, b):
            pltpu.make_async_copy(m_hbm.at[src_g.at[c]], bufs[b],
                                  semg[b]).wait()

        def s_start(c, b):
            pltpu.async_copy(bufs[b], agg_sh.at[dst_g.at[c]], sems[b],
                             add=True)

        def s_wait(c, b):
            pltpu.make_async_copy(bufs[b], agg_sh.at[dst_g.at[c]],
                                  sems[b]).wait()

        row0 = sid * rows_per_tile
        pltpu.sync_copy(zeros_hbm, agg_sh.at[pl.ds(row0, rows_per_tile)])
        plsc.subcore_barrier()

        for g in range(ngrp):
            pltpu.sync_copy(src_hbm.at[wid, g], src_g)
            pltpu.sync_copy(dst_hbm.at[wid, g], dst_g)

            g_start(0, 0)
            g_start(1, 1)
            g_start(2, 2)
            g_wait(0, 0)
            s_start(0, 0)
            g_start(3, 3)

            def slot(c, b):
                bf = (b + 3) % 4
                g_wait(c, b)
                s_wait(c - 1, bf)
                s_start(c, b)

                @pl.when(c + 3 < gchunk)
                def _():
                    g_start(c + 3, bf)

            def quad(t, carry):
                c0 = 1 + t * 4
                for o in range(4):
                    slot(c0 + o, (1 + o) % 4)
                return carry

            n_quads = (gchunk - 1) // 4
            lax.fori_loop(0, n_quads, quad, 0)
            for c in range(1 + 4 * n_quads, gchunk):
                slot(c, c % 4)
            s_wait(gchunk - 1, (gchunk - 1) % 4)

        plsc.subcore_barrier()
        pltpu.sync_copy(
            agg_sh.at[pl.ds(row0, rows_per_tile)],
            out_hbm.at[cid, pl.ds(row0, rows_per_tile)],
        )

    return seg_sum


def _post_body(p0_ref, p1_ref, x_ref, w_ref, wih_ref, whh_ref, bih_ref,
               bhh_ref, out_ref):
    d = x_ref.shape[1]
    xb = x_ref[...]
    w2 = lax.dot_general(w_ref[...], wih_ref[...], (((1,), (1,)), ((), ())),
                         preferred_element_type=jnp.float32)
    sx = p0_ref[0] + p1_ref[0]
    gi = (
        jnp.dot(sx, w2, preferred_element_type=jnp.float32)
        + bih_ref[...]
    )
    gh = (
        lax.dot_general(xb, whh_ref[...], (((1,), (1,)), ((), ())),
                        preferred_element_type=jnp.float32)
        + bhh_ref[...]
    )
    r = jax.nn.sigmoid(gi[:, :d] + gh[:, :d])
    z = jax.nn.sigmoid(gi[:, d:2 * d] + gh[:, d:2 * d])
    nn = jnp.tanh(gi[:, 2 * d:] + r * gh[:, 2 * d:])
    h = (1.0 - z) * nn + z * xb
    out_ref[...] = xb + jnp.maximum(h, 0.0)


def _post_call(part, x, w, wih, whh, bih, bhh, bn):
    n, d = x.shape
    d3 = wih.shape[0]
    grid = n // bn
    return pl.pallas_call(
        _post_body,
        grid=(grid,),
        in_specs=[
            pl.BlockSpec((1, bn, d), lambda i: (0, i, 0)),
            pl.BlockSpec((1, bn, d), lambda i: (1, i, 0)),
            pl.BlockSpec((bn, d), lambda i: (i, 0)),
            pl.BlockSpec((d, d), lambda i: (0, 0)),
            pl.BlockSpec((d3, d), lambda i: (0, 0)),
            pl.BlockSpec((d3, d), lambda i: (0, 0)),
            pl.BlockSpec((1, d3), lambda i: (0, 0)),
            pl.BlockSpec((1, d3), lambda i: (0, 0)),
        ],
        out_specs=pl.BlockSpec((bn, d), lambda i: (i, 0)),
        out_shape=jax.ShapeDtypeStruct((n, d), jnp.float32),
    )(part, part, x, w, wih, whh, bih, bhh)


def kernel(x, edge_index, W, W_ih, W_hh, b_ih, b_hh):
    n, d = x.shape
    e = edge_index.shape[1]
    src = edge_index[0]
    dst = edge_index[1]

    bn = 2000
    rows_per_tile = (-(-n // 16) + 7) // 8 * 8
    zeros = jnp.zeros((rows_per_tile, d), jnp.float32)

    nw, k, ngrp = 32, 80, 5
    e_pad = -(-e // (nw * ngrp * k)) * (nw * ngrp * k)
    gchunk = e_pad // (nw * ngrp * k)
    npad_rows = rows_per_tile * 16
    pad = e_pad - e
    if pad:
        pad_dst = n + jnp.arange(pad, dtype=jnp.int32) % (npad_rows - n)
        src_p = jnp.concatenate([src, jnp.zeros((pad,), jnp.int32)])
        dst_p = jnp.concatenate([dst, pad_dst])
    else:
        src_p, dst_p = src, dst
    src4 = src_p.reshape(nw, ngrp, gchunk, k)
    dst4 = dst_p.reshape(nw, ngrp, gchunk, k)
    part = _make_sc_seg_sum(n, d, e_pad, k, ngrp, gchunk)(x, src4, dst4, zeros)

    return _post_call(part, x, W, W_ih, W_hh, b_ih.reshape(1, -1),
                      b_hh.reshape(1, -1), bn)


# speedup vs baseline: 3.8331x; 1.0469x over previous
import functools

import jax
import jax.numpy as jnp
from jax import lax
from jax.experimental import pallas as pl
from jax.experimental.pallas import tpu as pltpu
from jax.experimental.pallas import tpu_sc as plsc


def _make_sc_seg_sum(n, d, e_pad, k, ngrp, gchunk):
    info = plsc.get_sparse_core_info()
    nc, ns = info.num_cores, info.num_subcores
    nw = nc * ns
    assert e_pad == nw * ngrp * gchunk * k
    rows_per_tile = (-(-n // ns) + 7) // 8 * 8
    npad = rows_per_tile * ns

    mesh = plsc.VectorSubcoreMesh(core_axis_name="c", subcore_axis_name="s")

    @functools.partial(
        pl.kernel,
        out_type=jax.ShapeDtypeStruct((nc, npad, d), jnp.float32),
        mesh=mesh,
        scratch_types=[
            pltpu.VMEM((gchunk, k), jnp.int32),
            pltpu.VMEM((gchunk, k), jnp.int32),
            pltpu.VMEM((k, d), jnp.float32),
            pltpu.VMEM((k, d), jnp.float32),
            pltpu.VMEM((k, d), jnp.float32),
            pltpu.VMEM((k, d), jnp.float32),
            pltpu.VMEM_SHARED((npad, d), jnp.float32),
            pltpu.SemaphoreType.DMA,
            pltpu.SemaphoreType.DMA,
            pltpu.SemaphoreType.DMA,
            pltpu.SemaphoreType.DMA,
            pltpu.SemaphoreType.DMA,
            pltpu.SemaphoreType.DMA,
            pltpu.SemaphoreType.DMA,
            pltpu.SemaphoreType.DMA,
        ],
    )
    def seg_sum(m_hbm, src_hbm, dst_hbm, out_hbm,
                src_g, dst_g, buf0, buf1, buf2, buf3, agg_sh,
                semg0, semg1, semg2, semg3, sems0, sems1, sems2, sems3):
        cid = lax.axis_index("c")
        sid = lax.axis_index("s")
        wid = sid * nc + cid
        bufs = (buf0, buf1, buf2, buf3)
        semg = (semg0, semg1, semg2, semg3)
        sems = (sems0, sems1, sems2, sems3)

        def g_start(c, b):
            pltpu.async_copy(m_hbm.at[src_g.at[c]], bufs[b], semg[b])

        def g_wait(c, b):
            pltpu.make_async_copy(m_hbm.at[src_g.at[c]], bufs[b],
                                  semg[b]).wait()

        def s_start(c, b):
            pltpu.async_copy(bufs[b], agg_sh.at[dst_g.at[c]], sems[b],
                             add=True)

        def s_wait(c, b):
            pltpu.make_async_copy(bufs[b], agg_sh.at[dst_g.at[c]],
                                  sems[b]).wait()

        row0 = sid * rows_per_tile

        for g in range(ngrp):
            pltpu.sync_copy(src_hbm.at[wid, g], src_g)
            pltpu.sync_copy(dst_hbm.at[wid, g], dst_g)

            g_start(0, 0)
            g_start(1, 1)
            g_start(2, 2)
            if g == 0:
                def zrow(i, carry):
                    for j in range(8):
                        buf3[i, pl.ds(j * 16, 16)] = jnp.zeros(
                            (16,), jnp.float32)
                    return carry

                lax.fori_loop(0, k, zrow, 0)
                nfull = rows_per_tile // k
                rem = rows_per_tile - nfull * k
                for t in range(nfull):
                    pltpu.sync_copy(
                        buf3, agg_sh.at[pl.ds(row0 + t * k, k)])
                if rem:
                    pltpu.sync_copy(
                        buf3.at[pl.ds(0, rem)],
                        agg_sh.at[pl.ds(row0 + nfull * k, rem)])
                plsc.subcore_barrier()
            g_wait(0, 0)
            s_start(0, 0)
            g_start(3, 3)

            def slot(c, b):
                bf = (b + 3) % 4
                g_wait(c, b)
                s_wait(c - 1, bf)
                s_start(c, b)

                @pl.when(c + 3 < gchunk)
                def _():
                    g_start(c + 3, bf)

            def quad(t, carry):
                c0 = 1 + t * 4
                for o in range(4):
                    slot(c0 + o, (1 + o) % 4)
                return carry

            n_quads = (gchunk - 1) // 4
            lax.fori_loop(0, n_quads, quad, 0)
            for c in range(1 + 4 * n_quads, gchunk):
                slot(c, c % 4)
            s_wait(gchunk - 1, (gchunk - 1) % 4)

        plsc.subcore_barrier()
        pltpu.sync_copy(
            agg_sh.at[pl.ds(row0, rows_per_tile)],
            out_hbm.at[cid, pl.ds(row0, rows_per_tile)],
        )

    return seg_sum


def _post_body(p0_ref, p1_ref, x_ref, w_ref, wih_ref, whh_ref, bih_ref,
               bhh_ref, out_ref):
    d = x_ref.shape[1]
    xb = x_ref[...]
    w2 = lax.dot_general(w_ref[...], wih_ref[...], (((1,), (1,)), ((), ())),
                         preferred_element_type=jnp.float32)
    sx = p0_ref[0] + p1_ref[0]
    gi = (
        jnp.dot(sx, w2, preferred_element_type=jnp.float32)
        + bih_ref[...]
    )
    gh = (
        lax.dot_general(xb, whh_ref[...], (((1,), (1,)), ((), ())),
                        preferred_element_type=jnp.float32)
        + bhh_ref[...]
    )
    r = jax.nn.sigmoid(gi[:, :d] + gh[:, :d])
    z = jax.nn.sigmoid(gi[:, d:2 * d] + gh[:, d:2 * d])
    nn = jnp.tanh(gi[:, 2 * d:] + r * gh[:, 2 * d:])
    h = (1.0 - z) * nn + z * xb
    out_ref[...] = xb + jnp.maximum(h, 0.0)


def _post_call(part, x, w, wih, whh, bih, bhh, bn):
    n, d = x.shape
    d3 = wih.shape[0]
    grid = n // bn
    return pl.pallas_call(
        _post_body,
        grid=(grid,),
        in_specs=[
            pl.BlockSpec((1, bn, d), lambda i: (0, i, 0)),
            pl.BlockSpec((1, bn, d), lambda i: (1, i, 0)),
            pl.BlockSpec((bn, d), lambda i: (i, 0)),
            pl.BlockSpec((d, d), lambda i: (0, 0)),
            pl.BlockSpec((d3, d), lambda i: (0, 0)),
            pl.BlockSpec((d3, d), lambda i: (0, 0)),
            pl.BlockSpec((1, d3), lambda i: (0, 0)),
            pl.BlockSpec((1, d3), lambda i: (0, 0)),
        ],
        out_specs=pl.BlockSpec((bn, d), lambda i: (i, 0)),
        out_shape=jax.ShapeDtypeStruct((n, d), jnp.float32),
    )(part, part, x, w, wih, whh, bih, bhh)


def kernel(x, edge_index, W, W_ih, W_hh, b_ih, b_hh):
    n, d = x.shape
    e = edge_index.shape[1]
    src = edge_index[0]
    dst = edge_index[1]

    bn = 2000
    rows_per_tile = (-(-n // 16) + 7) // 8 * 8

    nw, k, ngrp = 32, 80, 5
    e_pad = -(-e // (nw * ngrp * k)) * (nw * ngrp * k)
    gchunk = e_pad // (nw * ngrp * k)
    npad_rows = rows_per_tile * 16
    pad = e_pad - e
    if pad:
        pad_dst = n + jnp.arange(pad, dtype=jnp.int32) % (npad_rows - n)
        src_p = jnp.concatenate([src, jnp.zeros((pad,), jnp.int32)])
        dst_p = jnp.concatenate([dst, pad_dst])
    else:
        src_p, dst_p = src, dst
    src4 = src_p.reshape(nw, ngrp, gchunk, k)
    dst4 = dst_p.reshape(nw, ngrp, gchunk, k)
    part = _make_sc_seg_sum(n, d, e_pad, k, ngrp, gchunk)(x, src4, dst4)

    return _post_call(part, x, W, W_ih, W_hh, b_ih.reshape(1, -1),
                      b_hh.reshape(1, -1), bn)


# speedup vs baseline: 3.9231x; 1.0235x over previous
import functools

import jax
import jax.numpy as jnp
from jax import lax
from jax.experimental import pallas as pl
from jax.experimental.pallas import tpu as pltpu
from jax.experimental.pallas import tpu_sc as plsc


def _make_sc_seg_sum(n, d, e_pad, k, ngrp, gchunk):
    info = plsc.get_sparse_core_info()
    nc, ns = info.num_cores, info.num_subcores
    nw = nc * ns
    assert e_pad == nw * ngrp * gchunk * k
    rows_per_tile = (-(-n // ns) + 7) // 8 * 8
    npad = rows_per_tile * ns

    mesh = plsc.VectorSubcoreMesh(core_axis_name="c", subcore_axis_name="s")

    @functools.partial(
        pl.kernel,
        out_type=jax.ShapeDtypeStruct((nc, npad, d), jnp.float32),
        mesh=mesh,
        scratch_types=[
            pltpu.VMEM((gchunk, k), jnp.int32),
            pltpu.VMEM((gchunk, k), jnp.int32),
            pltpu.VMEM((k, d), jnp.float32),
            pltpu.VMEM((k, d), jnp.float32),
            pltpu.VMEM((k, d), jnp.float32),
            pltpu.VMEM((k, d), jnp.float32),
            pltpu.VMEM_SHARED((npad, d), jnp.float32),
            pltpu.SemaphoreType.DMA,
            pltpu.SemaphoreType.DMA,
            pltpu.SemaphoreType.DMA,
            pltpu.SemaphoreType.DMA,
            pltpu.SemaphoreType.DMA,
            pltpu.SemaphoreType.DMA,
            pltpu.SemaphoreType.DMA,
            pltpu.SemaphoreType.DMA,
        ],
    )
    def seg_sum(m_hbm, src_hbm, dst_hbm, out_hbm,
                src_g, dst_g, buf0, buf1, buf2, buf3, agg_sh,
                semg0, semg1, semg2, semg3, sems0, sems1, sems2, sems3):
        cid = lax.axis_index("c")
        sid = lax.axis_index("s")
        wid = sid * nc + cid
        bufs = (buf0, buf1, buf2, buf3)
        semg = (semg0, semg1, semg2, semg3)
        sems = (sems0, sems1, sems2, sems3)

        def g_start(c, b):
            pltpu.async_copy(m_hbm.at[src_g.at[c]], bufs[b], semg[b])

        def g_wait(c, b):
            pltpu.make_async_copy(m_hbm.at[src_g.at[c]], bufs[b],
                                  semg[b]).wait()

        def s_start(c, b):
            pltpu.async_copy(bufs[b], agg_sh.at[dst_g.at[c]], sems[b],
                             add=True)

        def s_wait(c, b):
            pltpu.make_async_copy(bufs[b], agg_sh.at[dst_g.at[c]],
                                  sems[b]).wait()

        row0 = sid * rows_per_tile

        for g in range(ngrp):
            cp_s = pltpu.async_copy(src_hbm.at[wid, g], src_g, sems[2])
            cp_d = pltpu.async_copy(dst_hbm.at[wid, g], dst_g, sems[3])
            cp_s.wait()

            g_start(0, 0)
            g_start(1, 1)
            g_start(2, 2)
            if g == 0:
                def zrow(i, carry):
                    for j in range(8):
                        buf3[i, pl.ds(j * 16, 16)] = jnp.zeros(
                            (16,), jnp.float32)
                    return carry

                lax.fori_loop(0, k, zrow, 0)
                nfull = rows_per_tile // k
                rem = rows_per_tile - nfull * k
                for t in range(nfull):
                    pltpu.sync_copy(
                        buf3, agg_sh.at[pl.ds(row0 + t * k, k)])
                if rem:
                    pltpu.sync_copy(
                        buf3.at[pl.ds(0, rem)],
                        agg_sh.at[pl.ds(row0 + nfull * k, rem)])
                plsc.subcore_barrier()
            cp_d.wait()
            g_wait(0, 0)
            s_start(0, 0)
            g_start(3, 3)

            def slot(c, b):
                bf = (b + 3) % 4
                g_wait(c, b)
                s_wait(c - 1, bf)
                s_start(c, b)

                @pl.when(c + 3 < gchunk)
                def _():
                    g_start(c + 3, bf)

            def quad(t, carry):
                c0 = 1 + t * 4
                for o in range(4):
                    slot(c0 + o, (1 + o) % 4)
                return carry

            n_quads = (gchunk - 1) // 4
            lax.fori_loop(0, n_quads, quad, 0)
            for c in range(1 + 4 * n_quads, gchunk):
                slot(c, c % 4)
            s_wait(gchunk - 1, (gchunk - 1) % 4)

        plsc.subcore_barrier()
        pltpu.sync_copy(
            agg_sh.at[pl.ds(row0, rows_per_tile)],
            out_hbm.at[cid, pl.ds(row0, rows_per_tile)],
        )

    return seg_sum


def _post_body(p0_ref, p1_ref, x_ref, w_ref, wih_ref, whh_ref, bih_ref,
               bhh_ref, out_ref):
    d = x_ref.shape[1]
    xb = x_ref[...]
    w2 = lax.dot_general(w_ref[...], wih_ref[...], (((1,), (1,)), ((), ())),
                         preferred_element_type=jnp.float32)
    sx = p0_ref[0] + p1_ref[0]
    gi = (
        jnp.dot(sx, w2, preferred_element_type=jnp.float32)
        + bih_ref[...]
    )
    gh = (
        lax.dot_general(xb, whh_ref[...], (((1,), (1,)), ((), ())),
                        preferred_element_type=jnp.float32)
        + bhh_ref[...]
    )
    r = jax.nn.sigmoid(gi[:, :d] + gh[:, :d])
    z = jax.nn.sigmoid(gi[:, d:2 * d] + gh[:, d:2 * d])
    nn = jnp.tanh(gi[:, 2 * d:] + r * gh[:, 2 * d:])
    h = (1.0 - z) * nn + z * xb
    out_ref[...] = xb + jnp.maximum(h, 0.0)


def _post_call(part, x, w, wih, whh, bih, bhh, bn):
    n, d = x.shape
    d3 = wih.shape[0]
    grid = n // bn
    return pl.pallas_call(
        _post_body,
        grid=(grid,),
        in_specs=[
            pl.BlockSpec((1, bn, d), lambda i: (0, i, 0)),
            pl.BlockSpec((1, bn, d), lambda i: (1, i, 0)),
            pl.BlockSpec((bn, d), lambda i: (i, 0)),
            pl.BlockSpec((d, d), lambda i: (0, 0)),
            pl.BlockSpec((d3, d), lambda i: (0, 0)),
            pl.BlockSpec((d3, d), lambda i: (0, 0)),
            pl.BlockSpec((1, d3), lambda i: (0, 0)),
            pl.BlockSpec((1, d3), lambda i: (0, 0)),
        ],
        out_specs=pl.BlockSpec((bn, d), lambda i: (i, 0)),
        out_shape=jax.ShapeDtypeStruct((n, d), jnp.float32),
    )(part, part, x, w, wih, whh, bih, bhh)


def kernel(x, edge_index, W, W_ih, W_hh, b_ih, b_hh):
    n, d = x.shape
    e = edge_index.shape[1]
    src = edge_index[0]
    dst = edge_index[1]

    bn = 2000
    rows_per_tile = (-(-n // 16) + 7) // 8 * 8

    nw, k, ngrp = 32, 80, 5
    e_pad = -(-e // (nw * ngrp * k)) * (nw * ngrp * k)
    gchunk = e_pad // (nw * ngrp * k)
    npad_rows = rows_per_tile * 16
    pad = e_pad - e
    if pad:
        pad_dst = n + jnp.arange(pad, dtype=jnp.int32) % (npad_rows - n)
        src_p = jnp.concatenate([src, jnp.zeros((pad,), jnp.int32)])
        dst_p = jnp.concatenate([dst, pad_dst])
    else:
        src_p, dst_p = src, dst
    src4 = src_p.reshape(nw, ngrp, gchunk, k)
    dst4 = dst_p.reshape(nw, ngrp, gchunk, k)
    part = _make_sc_seg_sum(n, d, e_pad, k, ngrp, gchunk)(x, src4, dst4)

    return _post_call(part, x, W, W_ih, W_hh, b_ih.reshape(1, -1),
                      b_hh.reshape(1, -1), bn)


# speedup vs baseline: 3.9267x; 1.0009x over previous
import functools

import jax
import jax.numpy as jnp
from jax import lax
from jax.experimental import pallas as pl
from jax.experimental.pallas import tpu as pltpu
from jax.experimental.pallas import tpu_sc as plsc


def _make_sc_seg_sum(n, d, e_pad, k, ngrp, gchunk):
    info = plsc.get_sparse_core_info()
    nc, ns = info.num_cores, info.num_subcores
    nw = nc * ns
    assert e_pad == nw * ngrp * gchunk * k
    rows_per_tile = (-(-n // ns) + 7) // 8 * 8
    npad = rows_per_tile * ns

    mesh = plsc.VectorSubcoreMesh(core_axis_name="c", subcore_axis_name="s")

    @functools.partial(
        pl.kernel,
        out_type=jax.ShapeDtypeStruct((nc, npad, d), jnp.float32),
        mesh=mesh,
        scratch_types=[
            pltpu.VMEM((gchunk, k), jnp.int32),
            pltpu.VMEM((gchunk, k), jnp.int32),
            pltpu.VMEM((k, d), jnp.float32),
            pltpu.VMEM((k, d), jnp.float32),
            pltpu.VMEM((k, d), jnp.float32),
            pltpu.VMEM((k, d), jnp.float32),
            pltpu.VMEM_SHARED((npad, d), jnp.float32),
            pltpu.SemaphoreType.DMA,
            pltpu.SemaphoreType.DMA,
            pltpu.SemaphoreType.DMA,
            pltpu.SemaphoreType.DMA,
            pltpu.SemaphoreType.DMA,
            pltpu.SemaphoreType.DMA,
            pltpu.SemaphoreType.DMA,
            pltpu.SemaphoreType.DMA,
        ],
    )
    def seg_sum(m_hbm, src_hbm, dst_hbm, out_hbm,
                src_g, dst_g, buf0, buf1, buf2, buf3, agg_sh,
                semg0, semg1, semg2, semg3, sems0, sems1, sems2, sems3):
        cid = lax.axis_index("c")
        sid = lax.axis_index("s")
        wid = sid * nc + cid
        bufs = (buf0, buf1, buf2, buf3)
        semg = (semg0, semg1, semg2, semg3)
        sems = (sems0, sems1, sems2, sems3)

        def g_start(c, b):
            pltpu.async_copy(m_hbm.at[src_g.at[c]], bufs[b], semg[b])

        def g_wait(c, b):
            pltpu.make_async_copy(m_hbm.at[src_g.at[c]], bufs[b],
                                  semg[b]).wait()

        def s_start(c, b):
            pltpu.async_copy(bufs[b], agg_sh.at[dst_g.at[c]], sems[b],
                             add=True)

        def s_wait(c, b):
            pltpu.make_async_copy(bufs[b], agg_sh.at[dst_g.at[c]],
                                  sems[b]).wait()

        row0 = sid * rows_per_tile

        for g in range(ngrp):
            cp_s = pltpu.async_copy(src_hbm.at[wid, g], src_g, sems[2])
            cp_d = pltpu.async_copy(dst_hbm.at[wid, g], dst_g, sems[3])
            cp_s.wait()

            g_start(0, 0)
            g_start(1, 1)
            g_start(2, 2)
            if g == 0:
                def zrow(i, carry):
                    for j in range(8):
                        buf3[i, pl.ds(j * 16, 16)] = jnp.zeros(
                            (16,), jnp.float32)
                    return carry

                lax.fori_loop(0, k, zrow, 0)
                nfull = rows_per_tile // k
                rem = rows_per_tile - nfull * k
                zcps = [
                    pltpu.async_copy(
                        buf3, agg_sh.at[pl.ds(row0 + t * k, k)], sems[2])
                    for t in range(nfull)
                ]
                if rem:
                    zcps.append(pltpu.async_copy(
                        buf3.at[pl.ds(0, rem)],
                        agg_sh.at[pl.ds(row0 + nfull * k, rem)], sems[2]))
                for cp in zcps:
                    cp.wait()
                plsc.subcore_barrier()
            cp_d.wait()
            g_wait(0, 0)
            s_start(0, 0)
            g_start(3, 3)

            def slot(c, b):
                bf = (b + 3) % 4
                g_wait(c, b)
                s_wait(c - 1, bf)
                s_start(c, b)

                @pl.when(c + 3 < gchunk)
                def _():
                    g_start(c + 3, bf)

            def quad(t, carry):
                c0 = 1 + t * 4
                for o in range(4):
                    slot(c0 + o, (1 + o) % 4)
                return carry

            n_quads = (gchunk - 1) // 4
            lax.fori_loop(0, n_quads, quad, 0)
            for c in range(1 + 4 * n_quads, gchunk):
                slot(c, c % 4)
            s_wait(gchunk - 1, (gchunk - 1) % 4)

        plsc.subcore_barrier()
        pltpu.sync_copy(
            agg_sh.at[pl.ds(row0, rows_per_tile)],
            out_hbm.at[cid, pl.ds(row0, rows_per_tile)],
        )

    return seg_sum


def _post_body(p0_ref, p1_ref, x_ref, w_ref, wih_ref, whh_ref, bih_ref,
               bhh_ref, out_ref):
    d = x_ref.shape[1]
    xb = x_ref[...]
    w2 = lax.dot_general(w_ref[...], wih_ref[...], (((1,), (1,)), ((), ())),
                         preferred_element_type=jnp.float32)
    sx = p0_ref[0] + p1_ref[0]
    gi = (
        jnp.dot(sx, w2, preferred_element_type=jnp.float32)
        + bih_ref[...]
    )
    gh = (
        lax.dot_general(xb, whh_ref[...], (((1,), (1,)), ((), ())),
                        preferred_element_type=jnp.float32)
        + bhh_ref[...]
    )
    r = jax.nn.sigmoid(gi[:, :d] + gh[:, :d])
    z = jax.nn.sigmoid(gi[:, d:2 * d] + gh[:, d:2 * d])
    nn = jnp.tanh(gi[:, 2 * d:] + r * gh[:, 2 * d:])
    h = (1.0 - z) * nn + z * xb
    out_ref[...] = xb + jnp.maximum(h, 0.0)


def _post_call(part, x, w, wih, whh, bih, bhh, bn):
    n, d = x.shape
    d3 = wih.shape[0]
    grid = n // bn
    return pl.pallas_call(
        _post_body,
        grid=(grid,),
        in_specs=[
            pl.BlockSpec((1, bn, d), lambda i: (0, i, 0)),
            pl.BlockSpec((1, bn, d), lambda i: (1, i, 0)),
            pl.BlockSpec((bn, d), lambda i: (i, 0)),
            pl.BlockSpec((d, d), lambda i: (0, 0)),
            pl.BlockSpec((d3, d), lambda i: (0, 0)),
            pl.BlockSpec((d3, d), lambda i: (0, 0)),
            pl.BlockSpec((1, d3), lambda i: (0, 0)),
            pl.BlockSpec((1, d3), lambda i: (0, 0)),
        ],
        out_specs=pl.BlockSpec((bn, d), lambda i: (i, 0)),
        out_shape=jax.ShapeDtypeStruct((n, d), jnp.float32),
    )(part, part, x, w, wih, whh, bih, bhh)


def kernel(x, edge_index, W, W_ih, W_hh, b_ih, b_hh):
    n, d = x.shape
    e = edge_index.shape[1]
    src = edge_index[0]
    dst = edge_index[1]

    bn = 2000
    rows_per_tile = (-(-n // 16) + 7) // 8 * 8

    nw, k, ngrp = 32, 80, 5
    e_pad = -(-e // (nw * ngrp * k)) * (nw * ngrp * k)
    gchunk = e_pad // (nw * ngrp * k)
    npad_rows = rows_per_tile * 16
    pad = e_pad - e
    if pad:
        pad_dst = n + jnp.arange(pad, dtype=jnp.int32) % (npad_rows - n)
        src_p = jnp.concatenate([src, jnp.zeros((pad,), jnp.int32)])
        dst_p = jnp.concatenate([dst, pad_dst])
    else:
        src_p, dst_p = src, dst
    src4 = src_p.reshape(nw, ngrp, gchunk, k)
    dst4 = dst_p.reshape(nw, ngrp, gchunk, k)
    part = _make_sc_seg_sum(n, d, e_pad, k, ngrp, gchunk)(x, src4, dst4)

    return _post_call(part, x, W, W_ih, W_hh, b_ih.reshape(1, -1),
                      b_hh.reshape(1, -1), bn)
